# Initial kernel scaffold; baseline (speedup 1.0000x reference)
import functools

import jax
import jax.numpy as jnp
from jax import lax
from jax.experimental import pallas as pl
from jax.experimental.pallas import tpu as pltpu
from jax.experimental.pallas import tpu_sc as plsc

_D = 64
_R = 8
_LANES = 16
_NDB = _D // _LANES
_NW = 32
_CHUNK = 128


def _make_sc_kernel(n_total):
    nchunk = n_total // (_NW * _CHUNK)

    mesh = plsc.VectorSubcoreMesh(core_axis_name="c", subcore_axis_name="s")

    @functools.partial(
        pl.kernel,
        mesh=mesh,
        out_type=jax.ShapeDtypeStruct((n_total, _D), jnp.float32),
        scratch_types=[
            pltpu.VMEM((nchunk, _CHUNK), jnp.int32),
            pltpu.VMEM((_CHUNK, _D), jnp.float32),
            pltpu.VMEM((_CHUNK, _R), jnp.float32),
            pltpu.VMEM((_R, _D), jnp.float32),
            pltpu.SemaphoreType.DMA,
        ],
    )
    def sc_kernel(idx_hbm, emb_hbm, a_hbm, b_hbm, out_hbm,
                  idx_v, rows_v, av_v, b_v, sem):
        num_cores = 2
        wid = lax.axis_index("s") * num_cores + lax.axis_index("c")

        pltpu.sync_copy(idx_hbm.at[wid], idx_v)
        pltpu.sync_copy(b_hbm, b_v)

        b_vecs = [[b_v[r, pl.ds(db * _LANES, _LANES)] for db in range(_NDB)]
                  for r in range(_R)]

        def chunk_body(j, carry):
            idx_row = idx_v.at[j]
            cp_e = pltpu.async_copy(emb_hbm.at[idx_row], rows_v, sem)
            cp_a = pltpu.async_copy(a_hbm.at[idx_row], av_v, sem)
            cp_e.wait()
            cp_a.wait()

            def row_body(i, c):
                avs = [jnp.broadcast_to(av_v[i, r], (_LANES,))
                       for r in range(_R)]
                for db in range(_NDB):
                    acc = rows_v[i, pl.ds(db * _LANES, _LANES)]
                    for r in range(_R):
                        acc = acc + avs[r] * b_vecs[r][db]
                    rows_v[i, pl.ds(db * _LANES, _LANES)] = acc
                return c

            lax.fori_loop(0, _CHUNK, row_body, 0)
            base = pl.multiple_of((wid * nchunk + j) * _CHUNK, _CHUNK)
            pltpu.sync_copy(rows_v, out_hbm.at[pl.ds(base, _CHUNK)])
            return carry

        lax.fori_loop(0, nchunk, chunk_body, 0)

    return sc_kernel


_N_TOTAL = 4096 * 50
_sc_kernel = _make_sc_kernel(_N_TOTAL)


def kernel(inputs, embeddings, lora_A, lora_B):
    batch, hist = inputs.shape
    idx = inputs.reshape(_NW, -1, _CHUNK)
    b_scaled = lora_B * jnp.sqrt(jnp.asarray(_D, jnp.float32))
    out = _sc_kernel(idx, embeddings, lora_A, b_scaled)
    return out.reshape(batch, hist, _D)


# speedup vs baseline: 2.7595x; 2.7595x over previous
import functools

import jax
import jax.numpy as jnp
from jax import lax
from jax.experimental import pallas as pl
from jax.experimental.pallas import tpu as pltpu
from jax.experimental.pallas import tpu_sc as plsc

_D = 64
_R = 8
_LANES = 16
_NDB = _D // _LANES
_NW = 32
_CHUNK = 128


def _make_sc_kernel(n_total):
    nchunk = n_total // (_NW * _CHUNK)

    mesh = plsc.VectorSubcoreMesh(core_axis_name="c", subcore_axis_name="s")

    @functools.partial(
        pl.kernel,
        mesh=mesh,
        compiler_params=pltpu.CompilerParams(use_tc_tiling_on_sc=False,
                                             needs_layout_passes=False),
        out_type=jax.ShapeDtypeStruct((n_total, _D), jnp.float32),
        scratch_types=[
            pltpu.VMEM((nchunk, _CHUNK), jnp.int32),
            pltpu.VMEM((_CHUNK, _D), jnp.float32),
            pltpu.VMEM((_CHUNK, _R), jnp.float32),
            pltpu.VMEM((_R, _D), jnp.float32),
            pltpu.SemaphoreType.DMA,
        ],
    )
    def sc_kernel(idx_hbm, emb_hbm, a_hbm, b_hbm, out_hbm,
                  idx_v, rows_v, av_v, b_v, sem):
        num_cores = 2
        wid = lax.axis_index("s") * num_cores + lax.axis_index("c")

        pltpu.sync_copy(idx_hbm.at[wid], idx_v)
        pltpu.sync_copy(b_hbm, b_v)

        b_vecs = [[b_v[r, pl.ds(db * _LANES, _LANES)] for db in range(_NDB)]
                  for r in range(_R)]

        def chunk_body(j, carry):
            idx_row = idx_v.at[j]
            cp_e = pltpu.async_copy(emb_hbm.at[idx_row], rows_v, sem)
            cp_a = pltpu.async_copy(a_hbm.at[idx_row], av_v, sem)
            cp_e.wait()
            cp_a.wait()

            def row_body(i, c):
                i_vec = jnp.full((_LANES,), i, jnp.int32)
                avs = [plsc.load_gather(
                           av_v, [i_vec, jnp.full((_LANES,), r, jnp.int32)])
                       for r in range(_R)]
                for db in range(_NDB):
                    acc = rows_v[i, pl.ds(db * _LANES, _LANES)]
                    for r in range(_R):
                        acc = acc + avs[r] * b_vecs[r][db]
                    rows_v[i, pl.ds(db * _LANES, _LANES)] = acc
                return c

            lax.fori_loop(0, _CHUNK, row_body, 0)
            base = pl.multiple_of((wid * nchunk + j) * _CHUNK, _CHUNK)
            pltpu.sync_copy(rows_v, out_hbm.at[pl.ds(base, _CHUNK)])
            return carry

        lax.fori_loop(0, nchunk, chunk_body, 0)

    return sc_kernel


_N_TOTAL = 4096 * 50
_sc_kernel = _make_sc_kernel(_N_TOTAL)


def kernel(inputs, embeddings, lora_A, lora_B):
    batch, hist = inputs.shape
    idx = inputs.reshape(_NW, -1, _CHUNK)
    b_scaled = lora_B * jnp.sqrt(jnp.asarray(_D, jnp.float32))
    out = _sc_kernel(idx, embeddings, lora_A, b_scaled)
    return out.reshape(batch, hist, _D)
